# baseline (speedup 1.0000x reference)
import jax
import jax.numpy as jnp
from jax.experimental import pallas as pl
from jax.experimental.pallas import tpu as pltpu

_B0 = 4000
_B1 = 4000


def _make_body(n, b0, b1):
    ns0 = n // b0
    ns1 = n // b1

    def body(x_ref, s_ref, o_hbm, ns_hbm, eb, ring, acc, scale, dsem):
        i = pl.program_id(0)

        def ns_copy(chunk, slot):
            return pltpu.make_async_copy(
                ring.at[slot, pl.ds(0, b0)], ns_hbm.at[pl.ds(chunk * b0, b0)],
                dsem.at[slot])

        def o_copy(chunk, slot):
            return pltpu.make_async_copy(
                ring.at[slot], o_hbm.at[pl.ds(chunk * b1, b1)], dsem.at[slot])

        @pl.when(i < ns0)
        def _phase0():
            slot = jax.lax.rem(i, 2)
            ns = x_ref[...] + s_ref[...]
            e = jnp.exp(ns)
            colsum = jnp.sum(e, axis=0, keepdims=True)
            acc[...] = jnp.where(i == 0, colsum, acc[...] + colsum)
            eb[pl.ds(i * b0, b0), :] = e.astype(jnp.bfloat16)

            @pl.when(i >= 2)
            def _drain():
                ns_copy(i - 2, slot).wait()

            ring[slot, pl.ds(0, b0)] = ns
            ns_copy(i, slot).start()

        @pl.when(i >= ns0)
        def _phase1():
            k = i - ns0
            slot = jax.lax.rem(k + ns0, 2)

            @pl.when(k == 0)
            def _transition():
                ns_copy(ns0 - 2, jax.lax.rem(ns0 - 2, 2)).wait()
                scale[...] = 2.0 / acc[...]

            @pl.when(k == 1)
            def _transition2():
                ns_copy(ns0 - 1, jax.lax.rem(ns0 - 1, 2)).wait()

            @pl.when(k >= 2)
            def _drain():
                o_copy(k - 2, slot).wait()

            ring[slot] = eb[pl.ds(k * b1, b1), :].astype(jnp.float32) * scale[...]
            o_copy(k, slot).start()

            @pl.when(k == ns1 - 1)
            def _epilogue():
                o_copy(k - 1, jax.lax.rem(k + ns0 + 1, 2)).wait()
                o_copy(k, slot).wait()

    return body


def kernel(x, state):
    xt = x.T
    st = state.T
    n, m = xt.shape
    b0, b1 = _B0, _B1
    ns0 = n // b0
    ns1 = n // b1
    in_spec = pl.BlockSpec((b0, m), lambda i: (jnp.minimum(i, ns0 - 1), 0))
    any_spec = pl.BlockSpec(memory_space=pl.ANY)
    out, ns = pl.pallas_call(
        _make_body(n, b0, b1),
        grid=(ns0 + ns1,),
        in_specs=[in_spec, in_spec],
        out_specs=[any_spec, any_spec],
        out_shape=[
            jax.ShapeDtypeStruct((n, m), xt.dtype),
            jax.ShapeDtypeStruct((n, m), xt.dtype),
        ],
        scratch_shapes=[
            pltpu.VMEM((n, m), jnp.bfloat16),
            pltpu.VMEM((2, b1, m), jnp.float32),
            pltpu.VMEM((1, m), jnp.float32),
            pltpu.VMEM((1, m), jnp.float32),
            pltpu.SemaphoreType.DMA((2,)),
        ],
    )(xt, st)
    return (out.T, ns.T)


# speedup vs baseline: 1.0372x; 1.0372x over previous
import jax
import jax.numpy as jnp
from jax.experimental import pallas as pl
from jax.experimental.pallas import tpu as pltpu

_B0 = 4000
_B1 = 10000


def _make_body(n, b0, b1):
    ns0 = n // b0
    ns1 = n // b1

    def body(x_ref, s_ref, o_hbm, ns_hbm, eb, ring, acc, scale, dsem):
        i = pl.program_id(0)

        def ns_copy(chunk, slot):
            return pltpu.make_async_copy(
                ring.at[slot, pl.ds(0, b0)], ns_hbm.at[pl.ds(chunk * b0, b0)],
                dsem.at[slot])

        def o_copy(chunk, slot):
            return pltpu.make_async_copy(
                ring.at[slot], o_hbm.at[pl.ds(chunk * b1, b1)], dsem.at[slot])

        @pl.when(i < ns0)
        def _phase0():
            slot = jax.lax.rem(i, 2)
            ns = x_ref[...] + s_ref[...]
            e = jnp.exp(ns)
            colsum = jnp.sum(e, axis=0, keepdims=True)
            acc[...] = jnp.where(i == 0, colsum, acc[...] + colsum)
            eb[pl.ds(i * b0, b0), :] = e.astype(jnp.bfloat16)

            @pl.when(i >= 2)
            def _drain():
                ns_copy(i - 2, slot).wait()

            ring[slot, pl.ds(0, b0)] = ns
            ns_copy(i, slot).start()

        @pl.when(i >= ns0)
        def _phase1():
            k = i - ns0
            slot = jax.lax.rem(k + ns0, 2)

            @pl.when(k == 0)
            def _transition():
                ns_copy(ns0 - 2, jax.lax.rem(ns0 - 2, 2)).wait()
                scale[...] = 2.0 / acc[...]

            @pl.when(k == 1)
            def _transition2():
                ns_copy(ns0 - 1, jax.lax.rem(ns0 - 1, 2)).wait()

            @pl.when(k >= 2)
            def _drain():
                o_copy(k - 2, slot).wait()

            ring[slot] = eb[pl.ds(k * b1, b1), :].astype(jnp.float32) * scale[...]
            o_copy(k, slot).start()

            @pl.when(k == ns1 - 1)
            def _epilogue():
                o_copy(k - 1, jax.lax.rem(k + ns0 + 1, 2)).wait()
                o_copy(k, slot).wait()

    return body


def kernel(x, state):
    xt = x.T
    st = state.T
    n, m = xt.shape
    b0, b1 = _B0, _B1
    ns0 = n // b0
    ns1 = n // b1
    in_spec = pl.BlockSpec((b0, m), lambda i: (jnp.minimum(i, ns0 - 1), 0))
    any_spec = pl.BlockSpec(memory_space=pl.ANY)
    out, ns = pl.pallas_call(
        _make_body(n, b0, b1),
        grid=(ns0 + ns1,),
        in_specs=[in_spec, in_spec],
        out_specs=[any_spec, any_spec],
        out_shape=[
            jax.ShapeDtypeStruct((n, m), xt.dtype),
            jax.ShapeDtypeStruct((n, m), xt.dtype),
        ],
        scratch_shapes=[
            pltpu.VMEM((n, m), jnp.bfloat16),
            pltpu.VMEM((2, b1, m), jnp.float32),
            pltpu.VMEM((1, m), jnp.float32),
            pltpu.VMEM((1, m), jnp.float32),
            pltpu.SemaphoreType.DMA((2,)),
        ],
    )(xt, st)
    return (out.T, ns.T)


# speedup vs baseline: 1.0442x; 1.0068x over previous
import jax
import jax.numpy as jnp
from jax.experimental import pallas as pl
from jax.experimental.pallas import tpu as pltpu

_B0 = 4000
_B1 = 10000


def _make_body(n, b0, b1):
    ns0 = n // b0
    ns1 = n // b1
    h0 = b0 // 2
    h1 = b1 // 2

    def body(xl_ref, xh_ref, sl_ref, sh_ref, o_hbm, ns_hbm,
             eb, ring, acc, scale, dsem):
        i = pl.program_id(0)

        def ns_copies(chunk, slot):
            return (
                pltpu.make_async_copy(
                    ring.at[slot, pl.ds(0, h0)],
                    ns_hbm.at[pl.ds(chunk * b0, h0)], dsem.at[slot, 0]),
                pltpu.make_async_copy(
                    ring.at[slot, pl.ds(h0, h0)],
                    ns_hbm.at[pl.ds(chunk * b0 + h0, h0)], dsem.at[slot, 1]),
            )

        def o_copies(chunk, slot):
            return (
                pltpu.make_async_copy(
                    ring.at[slot, pl.ds(0, h1)],
                    o_hbm.at[pl.ds(chunk * b1, h1)], dsem.at[slot, 0]),
                pltpu.make_async_copy(
                    ring.at[slot, pl.ds(h1, h1)],
                    o_hbm.at[pl.ds(chunk * b1 + h1, h1)], dsem.at[slot, 1]),
            )

        def start2(copies):
            copies[0].start()
            copies[1].start()

        def wait2(copies):
            copies[0].wait()
            copies[1].wait()

        @pl.when(i < ns0)
        def _phase0():
            slot = jax.lax.rem(i, 2)
            ns_lo = xl_ref[...] + sl_ref[...]
            ns_hi = xh_ref[...] + sh_ref[...]
            e_lo = jnp.exp(ns_lo)
            e_hi = jnp.exp(ns_hi)
            colsum = (jnp.sum(e_lo, axis=0, keepdims=True)
                      + jnp.sum(e_hi, axis=0, keepdims=True))
            acc[...] = jnp.where(i == 0, colsum, acc[...] + colsum)
            eb[pl.ds(i * b0, h0), :] = e_lo.astype(jnp.bfloat16)
            eb[pl.ds(i * b0 + h0, h0), :] = e_hi.astype(jnp.bfloat16)

            @pl.when(i >= 2)
            def _drain():
                wait2(ns_copies(i - 2, slot))

            ring[slot, pl.ds(0, h0)] = ns_lo
            ring[slot, pl.ds(h0, h0)] = ns_hi
            start2(ns_copies(i, slot))

        @pl.when(i >= ns0)
        def _phase1():
            k = i - ns0
            slot = jax.lax.rem(k + ns0, 2)

            @pl.when(k == 0)
            def _transition():
                wait2(ns_copies(ns0 - 2, jax.lax.rem(ns0 - 2, 2)))
                scale[...] = 2.0 / acc[...]

            @pl.when(k == 1)
            def _transition2():
                wait2(ns_copies(ns0 - 1, jax.lax.rem(ns0 - 1, 2)))

            @pl.when(k >= 2)
            def _drain():
                wait2(o_copies(k - 2, slot))

            ring[slot] = eb[pl.ds(k * b1, b1), :].astype(jnp.float32) * scale[...]
            start2(o_copies(k, slot))

            @pl.when(k == ns1 - 1)
            def _epilogue():
                wait2(o_copies(k - 1, jax.lax.rem(k + ns0 + 1, 2)))
                wait2(o_copies(k, slot))

    return body


def kernel(x, state):
    xt = x.T
    st = state.T
    n, m = xt.shape
    b0, b1 = _B0, _B1
    ns0 = n // b0
    ns1 = n // b1
    h0 = b0 // 2
    lo_spec = pl.BlockSpec(
        (h0, m), lambda i: (jnp.minimum(i, ns0 - 1) * 2, 0))
    hi_spec = pl.BlockSpec(
        (h0, m), lambda i: (jnp.minimum(i, ns0 - 1) * 2 + 1, 0))
    any_spec = pl.BlockSpec(memory_space=pl.ANY)
    out, ns = pl.pallas_call(
        _make_body(n, b0, b1),
        grid=(ns0 + ns1,),
        in_specs=[lo_spec, hi_spec, lo_spec, hi_spec],
        out_specs=[any_spec, any_spec],
        out_shape=[
            jax.ShapeDtypeStruct((n, m), xt.dtype),
            jax.ShapeDtypeStruct((n, m), xt.dtype),
        ],
        scratch_shapes=[
            pltpu.VMEM((n, m), jnp.bfloat16),
            pltpu.VMEM((2, b1, m), jnp.float32),
            pltpu.VMEM((1, m), jnp.float32),
            pltpu.VMEM((1, m), jnp.float32),
            pltpu.SemaphoreType.DMA((2, 2)),
        ],
    )(xt, xt, st, st)
    return (out.T, ns.T)


# speedup vs baseline: 1.0456x; 1.0014x over previous
import jax
import jax.numpy as jnp
from jax.experimental import pallas as pl
from jax.experimental.pallas import tpu as pltpu

_B0 = 4000
_B1 = 10000


def _make_body(n, b0, b1):
    ns0 = n // b0
    ns1 = n // b1
    h0 = b0 // 2
    h1 = b1 // 2

    def body(xl_ref, xh_ref, sl_ref, sh_ref, o_hbm, ns_hbm,
             eb, ring, acc, scale, dsem):
        i = pl.program_id(0)

        def ns_copies(chunk, slot):
            return (
                pltpu.make_async_copy(
                    ring.at[slot, pl.ds(0, h0)],
                    ns_hbm.at[pl.ds(chunk * b0, h0)], dsem.at[slot, 0]),
                pltpu.make_async_copy(
                    ring.at[slot, pl.ds(h0, h0)],
                    ns_hbm.at[pl.ds(chunk * b0 + h0, h0)], dsem.at[slot, 1]),
            )

        def o_copies(chunk, slot):
            q = b1 // 4
            return tuple(
                pltpu.make_async_copy(
                    ring.at[slot, pl.ds(p * q, q)],
                    o_hbm.at[pl.ds(chunk * b1 + p * q, q)], dsem.at[slot, p])
                for p in range(4))

        def start2(copies):
            for c in copies:
                c.start()

        def wait2(copies):
            for c in copies:
                c.wait()

        @pl.when(i < ns0)
        def _phase0():
            slot = jax.lax.rem(i, 2)
            ns_lo = xl_ref[...] + sl_ref[...]
            ns_hi = xh_ref[...] + sh_ref[...]
            e_lo = jnp.exp(ns_lo)
            e_hi = jnp.exp(ns_hi)
            colsum = (jnp.sum(e_lo, axis=0, keepdims=True)
                      + jnp.sum(e_hi, axis=0, keepdims=True))
            acc[...] = jnp.where(i == 0, colsum, acc[...] + colsum)
            eb[pl.ds(i * b0, h0), :] = e_lo.astype(jnp.bfloat16)
            eb[pl.ds(i * b0 + h0, h0), :] = e_hi.astype(jnp.bfloat16)

            @pl.when(i >= 2)
            def _drain():
                wait2(ns_copies(i - 2, slot))

            ring[slot, pl.ds(0, h0)] = ns_lo
            ring[slot, pl.ds(h0, h0)] = ns_hi
            start2(ns_copies(i, slot))

        @pl.when(i >= ns0)
        def _phase1():
            k = i - ns0
            slot = jax.lax.rem(k + ns0, 2)

            @pl.when(k == 0)
            def _transition():
                wait2(ns_copies(ns0 - 2, jax.lax.rem(ns0 - 2, 2)))
                scale[...] = 2.0 / acc[...]

            @pl.when(k == 1)
            def _transition2():
                wait2(ns_copies(ns0 - 1, jax.lax.rem(ns0 - 1, 2)))

            @pl.when(k >= 2)
            def _drain():
                wait2(o_copies(k - 2, slot))

            ring[slot] = eb[pl.ds(k * b1, b1), :].astype(jnp.float32) * scale[...]
            start2(o_copies(k, slot))

            @pl.when(k == ns1 - 1)
            def _epilogue():
                wait2(o_copies(k - 1, jax.lax.rem(k + ns0 + 1, 2)))
                wait2(o_copies(k, slot))

    return body


def kernel(x, state):
    xt = x.T
    st = state.T
    n, m = xt.shape
    b0, b1 = _B0, _B1
    ns0 = n // b0
    ns1 = n // b1
    h0 = b0 // 2
    lo_spec = pl.BlockSpec(
        (h0, m), lambda i: (jnp.minimum(i, ns0 - 1) * 2, 0))
    hi_spec = pl.BlockSpec(
        (h0, m), lambda i: (jnp.minimum(i, ns0 - 1) * 2 + 1, 0))
    any_spec = pl.BlockSpec(memory_space=pl.ANY)
    out, ns = pl.pallas_call(
        _make_body(n, b0, b1),
        grid=(ns0 + ns1,),
        in_specs=[lo_spec, hi_spec, lo_spec, hi_spec],
        out_specs=[any_spec, any_spec],
        out_shape=[
            jax.ShapeDtypeStruct((n, m), xt.dtype),
            jax.ShapeDtypeStruct((n, m), xt.dtype),
        ],
        scratch_shapes=[
            pltpu.VMEM((n, m), jnp.bfloat16),
            pltpu.VMEM((2, b1, m), jnp.float32),
            pltpu.VMEM((1, m), jnp.float32),
            pltpu.VMEM((1, m), jnp.float32),
            pltpu.SemaphoreType.DMA((2, 4)),
        ],
    )(xt, xt, st, st)
    return (out.T, ns.T)
